# baseline (speedup 1.0000x reference)
import functools

import jax
import jax.numpy as jnp
from jax import lax
from jax.experimental import pallas as pl
from jax.experimental.pallas import tpu as pltpu
from jax.experimental.pallas import tpu_sc as plsc

BATCH = 4096
HIST = 50
DIM = 64
NTOT = BATCH * HIST
NUM_CORES = 2
NUM_SUBCORES = 16
NW = NUM_CORES * NUM_SUBCORES
PER_W = NTOT // NW
CB = 128
NCH = PER_W // CB


UNROLL = 16


def _body(idx_hbm, table_hbm, out_hbm, idx_v, rows_v, trows_v, gsem):
    wid = lax.axis_index("s") * NUM_CORES + lax.axis_index("c")
    base = wid * PER_W
    pltpu.sync_copy(idx_hbm.at[pl.ds(base, PER_W)], idx_v)
    dlanes = lax.broadcasted_iota(jnp.int32, (16,), 0)
    dvecs = [d0 + dlanes for d0 in range(0, DIM, 16)]

    pltpu.async_copy(table_hbm.at[idx_v.at[pl.ds(0, CB)]], rows_v.at[0], gsem)

    def half_step(h, bank):
        pltpu.make_async_copy(
            table_hbm.at[pl.ds(0, CB)], rows_v.at[bank], gsem
        ).wait()

        @pl.when(h + 1 < NCH)
        def _():
            pltpu.async_copy(
                table_hbm.at[idx_v.at[pl.ds((h + 1) * CB, CB)]],
                rows_v.at[1 - bank],
                gsem,
            )

        @plsc.parallel_loop(0, CB, step=1, unroll=UNROLL)
        def _(b):
            bvec = jnp.full((16,), b, jnp.int32)
            for k, d0 in enumerate(range(0, DIM, 16)):
                vals = rows_v[bank, b, pl.ds(d0, 16)]
                plsc.store_scatter(trows_v, [dvecs[k], bvec], vals)
        pltpu.sync_copy(
            trows_v.at[pl.ds(0, DIM), pl.ds(0, CB)],
            out_hbm.at[h, pl.ds(0, DIM), pl.ds(wid * CB, CB)],
        )

    def step(hh, _):
        half_step(2 * hh, 0)
        half_step(2 * hh + 1, 1)
        return 0

    lax.fori_loop(0, NCH // 2, step, 0)


def kernel(x, table):
    idx = (
        x.astype(jnp.int32)
        .reshape(NW, CB, HIST)
        .transpose(0, 2, 1)
        .reshape(NTOT)
    )
    tab = jnp.pad(table, ((0, 0), (0, 128 - DIM)))
    mesh = plsc.VectorSubcoreMesh(core_axis_name="c", subcore_axis_name="s")
    run = functools.partial(
        pl.kernel,
        mesh=mesh,
        out_type=jax.ShapeDtypeStruct((HIST, DIM, BATCH), jnp.float32),
        scratch_types=[
            pltpu.VMEM((PER_W,), jnp.int32),
            pltpu.VMEM((2, CB, 128), jnp.float32),
            pltpu.VMEM((DIM, CB + 1), jnp.float32),
            pltpu.SemaphoreType.DMA,
        ],
        compiler_params=pltpu.CompilerParams(
            use_tc_tiling_on_sc=True,
            needs_layout_passes=False,
            disable_bounds_checks=True,
        ),
    )(_body)
    out = run(idx, tab)
    return jnp.transpose(out, (2, 0, 1))


# speedup vs baseline: 1.5087x; 1.5087x over previous
import functools

import jax
import jax.numpy as jnp
from jax import lax
from jax.experimental import pallas as pl
from jax.experimental.pallas import tpu as pltpu
from jax.experimental.pallas import tpu_sc as plsc

BATCH = 4096
HIST = 50
DIM = 64
NTOT = BATCH * HIST
NUM_CORES = 2
NUM_SUBCORES = 16
NW = NUM_CORES * NUM_SUBCORES
PER_W = NTOT // NW
CB = 128
NCH = PER_W // CB


UNROLL = 16


def _body(idx_hbm, table_hbm, out_hbm, idx_v, rows0_v, rows1_v, trows_v, gsem0, gsem1):
    wid = lax.axis_index("s") * NUM_CORES + lax.axis_index("c")
    base = wid * PER_W
    pltpu.sync_copy(idx_hbm.at[pl.ds(base, PER_W)], idx_v)
    dlanes = lax.broadcasted_iota(jnp.int32, (16,), 0)
    dvecs = [d0 + dlanes for d0 in range(0, DIM, 16)]
    perms = [(dlanes + s) & 15 for s in range(16)]

    pltpu.async_copy(table_hbm.at[idx_v.at[pl.ds(0, CB)]], rows0_v, gsem0)

    def half_step(h, mine_v, other_v, mysem, othersem):
        pltpu.make_async_copy(
            table_hbm.at[pl.ds(0, CB)], mine_v, mysem
        ).wait()

        @pl.when(h + 1 < NCH)
        def _():
            pltpu.async_copy(
                table_hbm.at[idx_v.at[pl.ds((h + 1) * CB, CB)]],
                other_v,
                othersem,
            )

        @plsc.parallel_loop(0, CB, step=16, unroll=2)
        def _(b0):
            rowv = b0 + dlanes
            for k in range(DIM // 16):
                for s in range(16):
                    colv = 16 * k + perms[s]
                    vals = plsc.load_gather(mine_v, [rowv, colv])
                    plsc.store_scatter(trows_v, [colv, rowv], vals)
        pltpu.sync_copy(
            trows_v, out_hbm.at[h, pl.ds(0, DIM), pl.ds(wid * CB, CB)]
        )

    def step(hh, _):
        half_step(2 * hh, rows0_v, rows1_v, gsem0, gsem1)
        half_step(2 * hh + 1, rows1_v, rows0_v, gsem1, gsem0)
        return 0

    lax.fori_loop(0, NCH // 2, step, 0)


def kernel(x, table):
    idx = (
        x.astype(jnp.int32)
        .reshape(NW, CB, HIST)
        .transpose(0, 2, 1)
        .reshape(NTOT)
    )
    tab = jnp.pad(table, ((0, 0), (0, 128 - DIM)))
    mesh = plsc.VectorSubcoreMesh(core_axis_name="c", subcore_axis_name="s")
    run = functools.partial(
        pl.kernel,
        mesh=mesh,
        out_type=jax.ShapeDtypeStruct((HIST, DIM, BATCH), jnp.float32),
        scratch_types=[
            pltpu.VMEM((PER_W,), jnp.int32),
            pltpu.VMEM((CB, 128), jnp.float32),
            pltpu.VMEM((CB, 128), jnp.float32),
            pltpu.VMEM((DIM, CB), jnp.float32),
            pltpu.SemaphoreType.DMA,
            pltpu.SemaphoreType.DMA,
        ],
        compiler_params=pltpu.CompilerParams(
            use_tc_tiling_on_sc=True,
            needs_layout_passes=False,
            disable_bounds_checks=True,
        ),
    )(_body)
    out = run(idx, tab)
    return jnp.transpose(out, (2, 0, 1))
